# baseline (speedup 1.0000x reference)
import functools

import numpy as np
import jax
import jax.numpy as jnp
from jax import lax
from jax.experimental import pallas as pl
from jax.experimental.pallas import tpu as pltpu
from jax.experimental.pallas import tpu_sc as plsc

_dim_used = np.array([6, 7, 8, 9, 10, 11, 12, 13, 14, 15, 16, 17, 21, 22, 23,
                      24, 25, 26, 27, 28, 29, 30, 31, 32, 36, 37, 38, 39, 40,
                      41, 42, 43, 44, 45, 46, 47, 51, 52, 53, 54, 55, 56, 57,
                      58, 59, 63, 64, 65, 66, 67, 68, 75, 76, 77, 78, 79, 80,
                      81, 82, 83, 87, 88, 89, 90, 91, 92])


def _joint_idx(x):
    return np.concatenate((x * 3, x * 3 + 1, x * 3 + 2))


_idx_copy = _joint_idx(np.array([0, 1, 6, 11]))
_idx_ignore = _joint_idx(np.array([16, 20, 23, 24, 28, 31]))
_idx_equal = _joint_idx(np.array([13, 19, 22, 13, 27, 30]))

_B, _T, _C, _P = 16384, 25, 96, 66
_PRED_ROW = _T * _P
_OBS_ROW = _C
_OUT_ROW = _T * _C
_NW = 32
_ROWS_PER_W = _B // _NW
_RB = 16
_NBLK = _ROWS_PER_W // _RB
_BUF_OBS_OFF = _RB * _PRED_ROW
_BUF_WORDS = _BUF_OBS_OFF + _RB * _OBS_ROW
_OUT_WORDS = _RB * _OUT_ROW


def _build_gather_table():
    src = [None] * _C
    for i, c in enumerate(_dim_used):
        src[int(c)] = ("pred", int(i))
    for c in _idx_copy:
        src[int(c)] = ("obs", int(c))
    for ig, eq in zip(_idx_ignore, _idx_equal):
        src[int(ig)] = src[int(eq)]
    assert all(s is not None for s in src)
    tab = np.empty((_RB, _T, _C), dtype=np.int32)
    for r in range(_RB):
        for t in range(_T):
            for c in range(_C):
                kind, j = src[c]
                if kind == "pred":
                    tab[r, t, c] = r * _PRED_ROW + t * _P + j
                else:
                    tab[r, t, c] = _BUF_OBS_OFF + r * _OBS_ROW + j
    return tab.reshape(-1)


_GTAB = _build_gather_table()


@functools.lru_cache(maxsize=1)
def _make_sc_call():
    mesh = plsc.VectorSubcoreMesh(core_axis_name="c", subcore_axis_name="s")

    @functools.partial(
        pl.kernel,
        mesh=mesh,
        out_type=jax.ShapeDtypeStruct((_B * _OUT_ROW,), jnp.float32),
        scratch_types=[
            pltpu.VMEM((_BUF_WORDS,), jnp.float32),
            pltpu.VMEM((_OUT_WORDS,), jnp.float32),
            pltpu.VMEM((_RB * _OUT_ROW,), jnp.int32),
        ],
        compiler_params=pltpu.CompilerParams(needs_layout_passes=False),
    )
    def _sc_postprocess(pred_hbm, obs_hbm, gtab_hbm, out_hbm, buf, out_v,
                        gtab_v):
        nc = 2
        wid = lax.axis_index("s") * nc + lax.axis_index("c")
        pltpu.sync_copy(gtab_hbm, gtab_v)

        def block(blk, carry):
            base = wid * _ROWS_PER_W + blk * _RB
            pltpu.sync_copy(
                pred_hbm.at[pl.ds(base * _PRED_ROW, _RB * _PRED_ROW)],
                buf.at[pl.ds(0, _RB * _PRED_ROW)])
            pltpu.sync_copy(
                obs_hbm.at[pl.ds(base * _OBS_ROW, _RB * _OBS_ROW)],
                buf.at[pl.ds(_BUF_OBS_OFF, _RB * _OBS_ROW)])

            def body(i, c2):
                tab = gtab_v[pl.ds(i * 16, 16)]
                out_v[pl.ds(i * 16, 16)] = plsc.load_gather(buf, [tab])
                return c2

            lax.fori_loop(0, _OUT_WORDS // 16, body, 0)
            pltpu.sync_copy(out_v,
                            out_hbm.at[pl.ds(base * _OUT_ROW, _OUT_WORDS)])
            return carry

        lax.fori_loop(0, _NBLK, block, 0)

    return _sc_postprocess


def kernel(observed_pose, pred_pose):
    b = pred_pose.shape[0]
    obs_last = observed_pose[:, -1, :].reshape(-1)
    pred_flat = pred_pose.reshape(-1)
    out = _make_sc_call()(pred_flat, obs_last, jnp.asarray(_GTAB))
    return out.reshape(b, _T, _C)


# speedup vs baseline: 6.9295x; 6.9295x over previous
import functools

import numpy as np
import jax
import jax.numpy as jnp
from jax import lax
from jax.experimental import pallas as pl
from jax.experimental.pallas import tpu as pltpu
from jax.experimental.pallas import tpu_sc as plsc

_dim_used = np.array([6, 7, 8, 9, 10, 11, 12, 13, 14, 15, 16, 17, 21, 22, 23,
                      24, 25, 26, 27, 28, 29, 30, 31, 32, 36, 37, 38, 39, 40,
                      41, 42, 43, 44, 45, 46, 47, 51, 52, 53, 54, 55, 56, 57,
                      58, 59, 63, 64, 65, 66, 67, 68, 75, 76, 77, 78, 79, 80,
                      81, 82, 83, 87, 88, 89, 90, 91, 92])


def _joint_idx(x):
    return np.concatenate((x * 3, x * 3 + 1, x * 3 + 2))


_idx_copy = _joint_idx(np.array([0, 1, 6, 11]))
_idx_ignore = _joint_idx(np.array([16, 20, 23, 24, 28, 31]))
_idx_equal = _joint_idx(np.array([13, 19, 22, 13, 27, 30]))

_B, _T, _C, _P = 16384, 25, 96, 66
_NW = 32
_CHK = 256
_NCHUNK = _B // (_NW * _CHK)
_KV = _CHK // 16


def _build_src():
    src = [None] * _C
    for i, c in enumerate(_dim_used):
        src[int(c)] = ("pred", int(i))
    for c in _idx_copy:
        src[int(c)] = ("obs", int(c))
    for ig, eq in zip(_idx_ignore, _idx_equal):
        src[int(ig)] = src[int(eq)]
    assert all(s is not None for s in src)
    return src


_SRC = _build_src()


@functools.lru_cache(maxsize=1)
def _make_sc_call():
    mesh = plsc.VectorSubcoreMesh(core_axis_name="c", subcore_axis_name="s")

    @functools.partial(
        pl.kernel,
        mesh=mesh,
        out_type=jax.ShapeDtypeStruct((_T, _C, _B), jnp.float32),
        scratch_types=[
            pltpu.VMEM((1, _P, _CHK), jnp.float32),
            pltpu.VMEM((1, _P, _CHK), jnp.float32),
            pltpu.VMEM((1, _C, _CHK), jnp.float32),
            pltpu.VMEM((1, _C, _CHK), jnp.float32),
            pltpu.VMEM((1, _C, _CHK), jnp.float32),
            pltpu.SemaphoreType.DMA,
            pltpu.SemaphoreType.DMA,
            pltpu.SemaphoreType.DMA,
            pltpu.SemaphoreType.DMA,
            pltpu.SemaphoreType.DMA,
        ],
        compiler_params=pltpu.CompilerParams(needs_layout_passes=False),
    )
    def _sc_postprocess(obs_hbm, pred_hbm, out_hbm, pred_v0, pred_v1, out_v0,
                        out_v1, obs_v, sem_i0, sem_i1, sem_o0, sem_o1,
                        sem_ob):
        nc = 2
        wid = lax.axis_index("s") * nc + lax.axis_index("c")
        pred_bufs = (pred_v0, pred_v1)
        out_bufs = (out_v0, out_v1)
        sem_in = (sem_i0, sem_i1)
        sem_out = (sem_o0, sem_o1)

        def in_cp(t, b0, buf):
            return pltpu.make_async_copy(
                pred_hbm.at[pl.ds(t, 1), :, pl.ds(b0, _CHK)],
                pred_bufs[buf], sem_in[buf])

        def out_cp(t, b0, buf):
            return pltpu.make_async_copy(
                out_bufs[buf],
                out_hbm.at[pl.ds(t, 1), :, pl.ds(b0, _CHK)], sem_out[buf])

        def assemble(pred_buf, out_buf):
            def kbody(k, carry):
                o = k * 16
                loaded = {}
                for c in range(_C):
                    kind, j = _SRC[c]
                    key = (kind, j)
                    if key not in loaded:
                        if kind == "pred":
                            loaded[key] = pred_buf[0, j, pl.ds(o, 16)]
                        else:
                            loaded[key] = obs_v[0, j, pl.ds(o, 16)]
                    out_buf[0, c, pl.ds(o, 16)] = loaded[key]
                return carry

            lax.fori_loop(0, _KV, kbody, 0)

        def chunk_body(ch, carry):
            b0 = wid * (_NCHUNK * _CHK) + ch * _CHK
            pltpu.make_async_copy(
                obs_hbm.at[pl.ds(_T * 2 - 1, 1), :, pl.ds(b0, _CHK)],
                obs_v, sem_ob).start()
            in_cp(0, b0, 0).start()
            pltpu.make_async_copy(
                obs_hbm.at[pl.ds(_T * 2 - 1, 1), :, pl.ds(b0, _CHK)],
                obs_v, sem_ob).wait()

            def pair_body(tt, c2):
                t0 = 2 * tt
                in_cp(t0, b0, 0).wait()
                in_cp(t0 + 1, b0, 1).start()

                @pl.when(tt > 0)
                def _():
                    out_cp(t0 - 2, b0, 0).wait()

                assemble(pred_bufs[0], out_bufs[0])
                out_cp(t0, b0, 0).start()
                in_cp(t0 + 1, b0, 1).wait()
                in_cp(t0 + 2, b0, 0).start()

                @pl.when(tt > 0)
                def _():
                    out_cp(t0 - 1, b0, 1).wait()

                assemble(pred_bufs[1], out_bufs[1])
                out_cp(t0 + 1, b0, 1).start()
                return c2

            lax.fori_loop(0, (_T - 1) // 2, pair_body, 0)
            t_last = _T - 1
            in_cp(t_last, b0, 0).wait()
            out_cp(t_last - 2, b0, 0).wait()
            assemble(pred_bufs[0], out_bufs[0])
            out_cp(t_last, b0, 0).start()
            out_cp(t_last - 1, b0, 1).wait()
            out_cp(t_last, b0, 0).wait()
            return carry

        lax.fori_loop(0, _NCHUNK, chunk_body, 0)

    return _sc_postprocess


def kernel(observed_pose, pred_pose):
    pred_r = jnp.transpose(pred_pose, (1, 2, 0))
    obs_r = jnp.transpose(observed_pose, (1, 2, 0))
    out_r = _make_sc_call()(obs_r, pred_r)
    return jnp.transpose(out_r, (2, 0, 1))


# speedup vs baseline: 6.9987x; 1.0100x over previous
import functools

import numpy as np
import jax
import jax.numpy as jnp
from jax import lax
from jax.experimental import pallas as pl
from jax.experimental.pallas import tpu as pltpu
from jax.experimental.pallas import tpu_sc as plsc

_dim_used = np.array([6, 7, 8, 9, 10, 11, 12, 13, 14, 15, 16, 17, 21, 22, 23,
                      24, 25, 26, 27, 28, 29, 30, 31, 32, 36, 37, 38, 39, 40,
                      41, 42, 43, 44, 45, 46, 47, 51, 52, 53, 54, 55, 56, 57,
                      58, 59, 63, 64, 65, 66, 67, 68, 75, 76, 77, 78, 79, 80,
                      81, 82, 83, 87, 88, 89, 90, 91, 92])


def _joint_idx(x):
    return np.concatenate((x * 3, x * 3 + 1, x * 3 + 2))


_idx_copy = _joint_idx(np.array([0, 1, 6, 11]))
_idx_ignore = _joint_idx(np.array([16, 20, 23, 24, 28, 31]))
_idx_equal = _joint_idx(np.array([13, 19, 22, 13, 27, 30]))

_B, _T, _C, _P = 16384, 25, 96, 66
_NW = 32
_CHK = 256
_NCHUNK = _B // (_NW * _CHK)
_KV = _CHK // 16
_OBSW = 40


def _build_src():
    src = [None] * _C
    for i, c in enumerate(_dim_used):
        src[int(c)] = ("pred", int(i))
    for c in _idx_copy:
        src[int(c)] = ("obs", int(c))
    for ig, eq in zip(_idx_ignore, _idx_equal):
        src[int(ig)] = src[int(eq)]
    assert all(s is not None for s in src)
    return src


_SRC = _build_src()


@functools.lru_cache(maxsize=1)
def _make_sc_call():
    mesh = plsc.VectorSubcoreMesh(core_axis_name="c", subcore_axis_name="s")

    @functools.partial(
        pl.kernel,
        mesh=mesh,
        out_type=jax.ShapeDtypeStruct((_T, _C, _B), jnp.float32),
        scratch_types=[
            pltpu.VMEM((1, _P, _CHK), jnp.float32),
            pltpu.VMEM((1, _P, _CHK), jnp.float32),
            pltpu.VMEM((1, _C, _CHK), jnp.float32),
            pltpu.VMEM((1, _C, _CHK), jnp.float32),
            pltpu.VMEM((1, _OBSW, _CHK), jnp.float32),
            pltpu.SemaphoreType.DMA,
            pltpu.SemaphoreType.DMA,
            pltpu.SemaphoreType.DMA,
            pltpu.SemaphoreType.DMA,
            pltpu.SemaphoreType.DMA,
        ],
        compiler_params=pltpu.CompilerParams(needs_layout_passes=False),
    )
    def _sc_postprocess(obs_hbm, pred_hbm, out_hbm, pred_v0, pred_v1, out_v0,
                        out_v1, obs_v, sem_i0, sem_i1, sem_o0, sem_o1,
                        sem_ob):
        nc = 2
        wid = lax.axis_index("s") * nc + lax.axis_index("c")
        pred_bufs = (pred_v0, pred_v1)
        out_bufs = (out_v0, out_v1)
        sem_in = (sem_i0, sem_i1)
        sem_out = (sem_o0, sem_o1)

        def in_cp(t, b0, buf):
            return pltpu.make_async_copy(
                pred_hbm.at[pl.ds(t, 1), :, pl.ds(b0, _CHK)],
                pred_bufs[buf], sem_in[buf])

        def out_cp(t, b0, buf):
            return pltpu.make_async_copy(
                out_bufs[buf],
                out_hbm.at[pl.ds(t, 1), :, pl.ds(b0, _CHK)], sem_out[buf])

        def assemble(pred_buf, out_buf):
            def kbody(k, carry):
                o = k * 16
                loaded = {}
                for c in range(_C):
                    kind, j = _SRC[c]
                    key = (kind, j)
                    if key not in loaded:
                        if kind == "pred":
                            loaded[key] = pred_buf[0, j, pl.ds(o, 16)]
                        else:
                            loaded[key] = obs_v[0, j, pl.ds(o, 16)]
                    out_buf[0, c, pl.ds(o, 16)] = loaded[key]
                return carry

            lax.fori_loop(0, _KV, kbody, 0)

        def obs_cp(b0):
            return pltpu.make_async_copy(
                obs_hbm.at[pl.ds(_T * 2 - 1, 1), pl.ds(0, _OBSW),
                           pl.ds(b0, _CHK)],
                obs_v, sem_ob)

        def chunk_base(ch):
            return wid * (_NCHUNK * _CHK) + ch * _CHK

        obs_cp(chunk_base(0)).start()
        in_cp(0, chunk_base(0), 0).start()

        def chunk_body(ch, carry):
            b0 = chunk_base(ch)
            obs_cp(b0).wait()

            def pair_body(tt, c2):
                t0 = 2 * tt
                in_cp(t0, b0, 0).wait()
                in_cp(t0 + 1, b0, 1).start()

                @pl.when(tt > 0)
                def _():
                    out_cp(t0 - 2, b0, 0).wait()

                assemble(pred_bufs[0], out_bufs[0])
                out_cp(t0, b0, 0).start()
                in_cp(t0 + 1, b0, 1).wait()
                in_cp(t0 + 2, b0, 0).start()

                @pl.when(tt > 0)
                def _():
                    out_cp(t0 - 1, b0, 1).wait()

                assemble(pred_bufs[1], out_bufs[1])
                out_cp(t0 + 1, b0, 1).start()
                return c2

            lax.fori_loop(0, (_T - 1) // 2, pair_body, 0)
            t_last = _T - 1
            in_cp(t_last, b0, 0).wait()
            out_cp(t_last - 2, b0, 0).wait()
            assemble(pred_bufs[0], out_bufs[0])
            out_cp(t_last, b0, 0).start()

            @pl.when(ch + 1 < _NCHUNK)
            def _():
                obs_cp(chunk_base(ch + 1)).start()
                in_cp(0, chunk_base(ch + 1), 0).start()

            out_cp(t_last - 1, b0, 1).wait()
            out_cp(t_last, b0, 0).wait()
            return carry

        lax.fori_loop(0, _NCHUNK, chunk_body, 0)

    return _sc_postprocess


def kernel(observed_pose, pred_pose):
    pred_r = jnp.transpose(pred_pose, (1, 2, 0))
    obs_r = jnp.transpose(observed_pose, (1, 2, 0))
    out_r = _make_sc_call()(obs_r, pred_r)
    return jnp.transpose(out_r, (2, 0, 1))


# speedup vs baseline: 7.9890x; 1.1415x over previous
import functools

import numpy as np
import jax
import jax.numpy as jnp
from jax import lax
from jax.experimental import pallas as pl
from jax.experimental.pallas import tpu as pltpu
from jax.experimental.pallas import tpu_sc as plsc

_dim_used = np.array([6, 7, 8, 9, 10, 11, 12, 13, 14, 15, 16, 17, 21, 22, 23,
                      24, 25, 26, 27, 28, 29, 30, 31, 32, 36, 37, 38, 39, 40,
                      41, 42, 43, 44, 45, 46, 47, 51, 52, 53, 54, 55, 56, 57,
                      58, 59, 63, 64, 65, 66, 67, 68, 75, 76, 77, 78, 79, 80,
                      81, 82, 83, 87, 88, 89, 90, 91, 92])


def _joint_idx(x):
    return np.concatenate((x * 3, x * 3 + 1, x * 3 + 2))


_idx_copy = _joint_idx(np.array([0, 1, 6, 11]))
_idx_ignore = _joint_idx(np.array([16, 20, 23, 24, 28, 31]))
_idx_equal = _joint_idx(np.array([13, 19, 22, 13, 27, 30]))

_B, _T, _C, _P = 16384, 25, 96, 66
_NW = 32
_CHK = 256
_NCHUNK = _B // (_NW * _CHK)
_KV = _CHK // 16
_OBSW = 24
_OBS_ROWMAP = {**{c: c for c in range(8)},
               **{c: c - 8 for c in range(16, 24)},
               **{c: c - 16 for c in range(32, 40)}}


def _build_src():
    src = [None] * _C
    for i, c in enumerate(_dim_used):
        src[int(c)] = ("pred", int(i))
    for c in _idx_copy:
        src[int(c)] = ("obs", int(c))
    for ig, eq in zip(_idx_ignore, _idx_equal):
        src[int(ig)] = src[int(eq)]
    assert all(s is not None for s in src)
    return src


_SRC = _build_src()


@functools.lru_cache(maxsize=1)
def _make_sc_call():
    mesh = plsc.VectorSubcoreMesh(core_axis_name="c", subcore_axis_name="s")

    @functools.partial(
        pl.kernel,
        mesh=mesh,
        out_type=jax.ShapeDtypeStruct((_T, _C, _B), jnp.float32),
        scratch_types=[
            pltpu.VMEM((2, _P, _CHK), jnp.float32),
            pltpu.VMEM((2, _P, _CHK), jnp.float32),
            pltpu.VMEM((1, _C, _CHK), jnp.float32),
            pltpu.VMEM((1, _C, _CHK), jnp.float32),
            pltpu.VMEM((1, _OBSW, _CHK), jnp.float32),
            pltpu.SemaphoreType.DMA,
            pltpu.SemaphoreType.DMA,
            pltpu.SemaphoreType.DMA,
            pltpu.SemaphoreType.DMA,
            pltpu.SemaphoreType.DMA,
        ],
        compiler_params=pltpu.CompilerParams(needs_layout_passes=False),
    )
    def _sc_postprocess(obs_hbm, pred_hbm, out_hbm, pred_v0, pred_v1, out_v0,
                        out_v1, obs_v, sem_i0, sem_i1, sem_o0, sem_o1,
                        sem_ob):
        nc = 2
        wid = lax.axis_index("s") * nc + lax.axis_index("c")
        pred_bufs = (pred_v0, pred_v1)
        out_bufs = (out_v0, out_v1)
        sem_in = (sem_i0, sem_i1)
        sem_out = (sem_o0, sem_o1)

        def in_cp(p, b0, buf):
            return pltpu.make_async_copy(
                pred_hbm.at[pl.ds(2 * p, 2), :, pl.ds(b0, _CHK)],
                pred_bufs[buf], sem_in[buf])

        def in_tail_cp(b0):
            return pltpu.make_async_copy(
                pred_hbm.at[pl.ds(_T - 1, 1), :, pl.ds(b0, _CHK)],
                pred_bufs[0].at[pl.ds(0, 1)], sem_in[0])

        def out_cp(t, b0, buf):
            return pltpu.make_async_copy(
                out_bufs[buf],
                out_hbm.at[pl.ds(t, 1), :, pl.ds(b0, _CHK)], sem_out[buf])

        def assemble(pred_buf, tsub, out_buf):
            def kbody(k, carry):
                o = k * 16
                loaded = {}
                for c in range(_C):
                    kind, j = _SRC[c]
                    key = (kind, j)
                    if key not in loaded:
                        if kind == "pred":
                            loaded[key] = pred_buf[tsub, j, pl.ds(o, 16)]
                        else:
                            loaded[key] = obs_v[0, _OBS_ROWMAP[j],
                                                pl.ds(o, 16)]
                    out_buf[0, c, pl.ds(o, 16)] = loaded[key]
                return carry

            lax.fori_loop(0, _KV, kbody, 0)

        def obs_cps(b0):
            return [
                pltpu.make_async_copy(
                    obs_hbm.at[pl.ds(_T * 2 - 1, 1), pl.ds(16 * g, 8),
                               pl.ds(b0, _CHK)],
                    obs_v.at[:, pl.ds(8 * g, 8)], sem_ob)
                for g in range(3)
            ]

        def chunk_base(ch):
            return wid * (_NCHUNK * _CHK) + ch * _CHK

        for cp in obs_cps(chunk_base(0)):
            cp.start()
        in_cp(0, chunk_base(0), 0).start()

        n_pairs = (_T - 1) // 2

        def chunk_body(ch, carry):
            b0 = chunk_base(ch)
            for cp in obs_cps(b0):
                cp.wait()

            def quad_body(pq, c2):
                p0 = 2 * pq
                in_cp(p0, b0, 0).wait()
                in_cp(p0 + 1, b0, 1).start()

                @pl.when(pq > 0)
                def _():
                    out_cp(2 * p0 - 2, b0, 0).wait()

                assemble(pred_bufs[0], 0, out_bufs[0])
                out_cp(2 * p0, b0, 0).start()

                @pl.when(pq > 0)
                def _():
                    out_cp(2 * p0 - 1, b0, 1).wait()

                assemble(pred_bufs[0], 1, out_bufs[1])
                out_cp(2 * p0 + 1, b0, 1).start()

                p1 = p0 + 1
                in_cp(p1, b0, 1).wait()

                @pl.when(pq < n_pairs // 2 - 1)
                def _():
                    in_cp(p1 + 1, b0, 0).start()

                @pl.when(pq == n_pairs // 2 - 1)
                def _():
                    in_tail_cp(b0).start()

                out_cp(2 * p1 - 2, b0, 0).wait()
                assemble(pred_bufs[1], 0, out_bufs[0])
                out_cp(2 * p1, b0, 0).start()
                out_cp(2 * p1 - 1, b0, 1).wait()
                assemble(pred_bufs[1], 1, out_bufs[1])
                out_cp(2 * p1 + 1, b0, 1).start()
                return c2

            lax.fori_loop(0, n_pairs // 2, quad_body, 0)
            t_last = _T - 1
            in_tail_cp(b0).wait()
            out_cp(t_last - 2, b0, 0).wait()
            assemble(pred_bufs[0], 0, out_bufs[0])
            out_cp(t_last, b0, 0).start()

            @pl.when(ch + 1 < _NCHUNK)
            def _():
                for cp in obs_cps(chunk_base(ch + 1)):
                    cp.start()
                in_cp(0, chunk_base(ch + 1), 0).start()

            out_cp(t_last - 1, b0, 1).wait()
            out_cp(t_last, b0, 0).wait()
            return carry

        lax.fori_loop(0, _NCHUNK, chunk_body, 0)

    return _sc_postprocess


def kernel(observed_pose, pred_pose):
    pred_r = jnp.transpose(pred_pose, (1, 2, 0))
    obs_r = jnp.transpose(observed_pose, (1, 2, 0))
    out_r = _make_sc_call()(obs_r, pred_r)
    return jnp.transpose(out_r, (2, 0, 1))


# speedup vs baseline: 9.6032x; 1.2021x over previous
import functools

import numpy as np
import jax
import jax.numpy as jnp
from jax import lax
from jax.experimental import pallas as pl
from jax.experimental.pallas import tpu as pltpu
from jax.experimental.pallas import tpu_sc as plsc

_dim_used = np.array([6, 7, 8, 9, 10, 11, 12, 13, 14, 15, 16, 17, 21, 22, 23,
                      24, 25, 26, 27, 28, 29, 30, 31, 32, 36, 37, 38, 39, 40,
                      41, 42, 43, 44, 45, 46, 47, 51, 52, 53, 54, 55, 56, 57,
                      58, 59, 63, 64, 65, 66, 67, 68, 75, 76, 77, 78, 79, 80,
                      81, 82, 83, 87, 88, 89, 90, 91, 92])


def _joint_idx(x):
    return np.concatenate((x * 3, x * 3 + 1, x * 3 + 2))


_idx_copy = _joint_idx(np.array([0, 1, 6, 11]))
_idx_ignore = _joint_idx(np.array([16, 20, 23, 24, 28, 31]))
_idx_equal = _joint_idx(np.array([13, 19, 22, 13, 27, 30]))

_B, _T, _C, _P = 16384, 25, 96, 66
_NW = 32
_CHK = 256
_NCHUNK = _B // (_NW * _CHK)
_KV = _CHK // 16
_OBSW = 24
_OBS_ROWMAP = {**{c: c for c in range(8)},
               **{c: c - 8 for c in range(16, 24)},
               **{c: c - 16 for c in range(32, 40)}}


def _build_src():
    src = [None] * _C
    for i, c in enumerate(_dim_used):
        src[int(c)] = ("pred", int(i))
    for c in _idx_copy:
        src[int(c)] = ("obs", int(c))
    for ig, eq in zip(_idx_ignore, _idx_equal):
        src[int(ig)] = src[int(eq)]
    assert all(s is not None for s in src)
    return src


_SRC = _build_src()


@functools.lru_cache(maxsize=1)
def _make_sc_call():
    mesh = plsc.VectorSubcoreMesh(core_axis_name="c", subcore_axis_name="s")

    @functools.partial(
        pl.kernel,
        mesh=mesh,
        out_type=jax.ShapeDtypeStruct((_T, _C, _B), jnp.float32),
        scratch_types=[
            pltpu.VMEM((2, _P, _CHK), jnp.float32),
            pltpu.VMEM((2, _P, _CHK), jnp.float32),
            pltpu.VMEM((1, _C, _CHK), jnp.float32),
            pltpu.VMEM((1, _C, _CHK), jnp.float32),
            pltpu.VMEM((1, _OBSW, _CHK), jnp.float32),
            pltpu.SemaphoreType.DMA,
            pltpu.SemaphoreType.DMA,
            pltpu.SemaphoreType.DMA,
            pltpu.SemaphoreType.DMA,
            pltpu.SemaphoreType.DMA,
        ],
        compiler_params=pltpu.CompilerParams(needs_layout_passes=False),
    )
    def _sc_postprocess(obs_hbm, pred_hbm, out_hbm, pred_v0, pred_v1, out_v0,
                        out_v1, obs_v, sem_i0, sem_i1, sem_o0, sem_o1,
                        sem_ob):
        nc = 2
        wid = lax.axis_index("s") * nc + lax.axis_index("c")
        pred_bufs = (pred_v0, pred_v1)
        out_bufs = (out_v0, out_v1)
        sem_in = (sem_i0, sem_i1)
        sem_out = (sem_o0, sem_o1)

        def in_cp(p, b0, buf):
            return pltpu.make_async_copy(
                pred_hbm.at[pl.ds(2 * p, 2), :, pl.ds(b0, _CHK)],
                pred_bufs[buf], sem_in[buf])

        def in_tail_cp(b0):
            return pltpu.make_async_copy(
                pred_hbm.at[pl.ds(_T - 1, 1), :, pl.ds(b0, _CHK)],
                pred_bufs[0].at[pl.ds(0, 1)], sem_in[0])

        def out_cp(t, b0, buf):
            return pltpu.make_async_copy(
                out_bufs[buf],
                out_hbm.at[pl.ds(t, 1), :, pl.ds(b0, _CHK)], sem_out[buf])

        def assemble(pred_buf, tsub, out_buf):
            @plsc.parallel_loop(0, _CHK, 16, unroll=2)
            def kbody(o):
                loaded = {}
                for c in range(_C):
                    kind, j = _SRC[c]
                    key = (kind, j)
                    if key not in loaded:
                        if kind == "pred":
                            loaded[key] = pred_buf[tsub, j, pl.ds(o, 16)]
                        else:
                            loaded[key] = obs_v[0, _OBS_ROWMAP[j],
                                                pl.ds(o, 16)]
                    out_buf[0, c, pl.ds(o, 16)] = loaded[key]

        def obs_cps(b0):
            return [
                pltpu.make_async_copy(
                    obs_hbm.at[pl.ds(_T * 2 - 1, 1), pl.ds(16 * g, 8),
                               pl.ds(b0, _CHK)],
                    obs_v.at[:, pl.ds(8 * g, 8)], sem_ob)
                for g in range(3)
            ]

        def chunk_base(ch):
            return wid * (_NCHUNK * _CHK) + ch * _CHK

        for cp in obs_cps(chunk_base(0)):
            cp.start()
        in_cp(0, chunk_base(0), 0).start()

        n_pairs = (_T - 1) // 2

        def chunk_body(ch, carry):
            b0 = chunk_base(ch)
            for cp in obs_cps(b0):
                cp.wait()

            def quad_body(pq, c2):
                p0 = 2 * pq
                in_cp(p0, b0, 0).wait()
                in_cp(p0 + 1, b0, 1).start()

                @pl.when(pq > 0)
                def _():
                    out_cp(2 * p0 - 2, b0, 0).wait()

                assemble(pred_bufs[0], 0, out_bufs[0])
                out_cp(2 * p0, b0, 0).start()

                @pl.when(pq > 0)
                def _():
                    out_cp(2 * p0 - 1, b0, 1).wait()

                assemble(pred_bufs[0], 1, out_bufs[1])
                out_cp(2 * p0 + 1, b0, 1).start()

                p1 = p0 + 1
                in_cp(p1, b0, 1).wait()

                @pl.when(pq < n_pairs // 2 - 1)
                def _():
                    in_cp(p1 + 1, b0, 0).start()

                @pl.when(pq == n_pairs // 2 - 1)
                def _():
                    in_tail_cp(b0).start()

                out_cp(2 * p1 - 2, b0, 0).wait()
                assemble(pred_bufs[1], 0, out_bufs[0])
                out_cp(2 * p1, b0, 0).start()
                out_cp(2 * p1 - 1, b0, 1).wait()
                assemble(pred_bufs[1], 1, out_bufs[1])
                out_cp(2 * p1 + 1, b0, 1).start()
                return c2

            lax.fori_loop(0, n_pairs // 2, quad_body, 0)
            t_last = _T - 1
            in_tail_cp(b0).wait()
            out_cp(t_last - 2, b0, 0).wait()
            assemble(pred_bufs[0], 0, out_bufs[0])
            out_cp(t_last, b0, 0).start()

            @pl.when(ch + 1 < _NCHUNK)
            def _():
                for cp in obs_cps(chunk_base(ch + 1)):
                    cp.start()
                in_cp(0, chunk_base(ch + 1), 0).start()

            out_cp(t_last - 1, b0, 1).wait()
            out_cp(t_last, b0, 0).wait()
            return carry

        lax.fori_loop(0, _NCHUNK, chunk_body, 0)

    return _sc_postprocess


def kernel(observed_pose, pred_pose):
    pred_r = jnp.transpose(pred_pose, (1, 2, 0))
    obs_r = jnp.transpose(observed_pose, (1, 2, 0))
    out_r = _make_sc_call()(obs_r, pred_r)
    return jnp.transpose(out_r, (2, 0, 1))


# speedup vs baseline: 10.4834x; 1.0917x over previous
import functools

import numpy as np
import jax
import jax.numpy as jnp
from jax import lax
from jax.experimental import pallas as pl
from jax.experimental.pallas import tpu as pltpu
from jax.experimental.pallas import tpu_sc as plsc

_dim_used = np.array([6, 7, 8, 9, 10, 11, 12, 13, 14, 15, 16, 17, 21, 22, 23,
                      24, 25, 26, 27, 28, 29, 30, 31, 32, 36, 37, 38, 39, 40,
                      41, 42, 43, 44, 45, 46, 47, 51, 52, 53, 54, 55, 56, 57,
                      58, 59, 63, 64, 65, 66, 67, 68, 75, 76, 77, 78, 79, 80,
                      81, 82, 83, 87, 88, 89, 90, 91, 92])


def _joint_idx(x):
    return np.concatenate((x * 3, x * 3 + 1, x * 3 + 2))


_idx_copy = _joint_idx(np.array([0, 1, 6, 11]))
_idx_ignore = _joint_idx(np.array([16, 20, 23, 24, 28, 31]))
_idx_equal = _joint_idx(np.array([13, 19, 22, 13, 27, 30]))

_B, _T, _C, _P = 16384, 25, 96, 66
_NW = 32
_CHK = 256
_NCHUNK = _B // (_NW * _CHK)
_KV = _CHK // 16
_OBSW = 24
_OBS_ROWMAP = {**{c: c for c in range(8)},
               **{c: c - 8 for c in range(16, 24)},
               **{c: c - 16 for c in range(32, 40)}}


def _build_src():
    src = [None] * _C
    for i, c in enumerate(_dim_used):
        src[int(c)] = ("pred", int(i))
    for c in _idx_copy:
        src[int(c)] = ("obs", int(c))
    for ig, eq in zip(_idx_ignore, _idx_equal):
        src[int(ig)] = src[int(eq)]
    assert all(s is not None for s in src)
    return src


_SRC = _build_src()


@functools.lru_cache(maxsize=1)
def _make_sc_call():
    mesh = plsc.VectorSubcoreMesh(core_axis_name="c", subcore_axis_name="s")

    @functools.partial(
        pl.kernel,
        mesh=mesh,
        out_type=jax.ShapeDtypeStruct((_T, _C, _B), jnp.float32),
        scratch_types=[
            pltpu.VMEM((2, _P, _CHK), jnp.float32),
            pltpu.VMEM((2, _P, _CHK), jnp.float32),
            pltpu.VMEM((1, _C, _CHK), jnp.float32),
            pltpu.VMEM((1, _C, _CHK), jnp.float32),
            pltpu.VMEM((1, _OBSW, _CHK), jnp.float32),
            pltpu.SemaphoreType.DMA,
            pltpu.SemaphoreType.DMA,
            pltpu.SemaphoreType.DMA,
            pltpu.SemaphoreType.DMA,
            pltpu.SemaphoreType.DMA,
        ],
        compiler_params=pltpu.CompilerParams(needs_layout_passes=False),
    )
    def _sc_postprocess(obs_hbm, pred_hbm, out_hbm, pred_v0, pred_v1, out_v0,
                        out_v1, obs_v, sem_i0, sem_i1, sem_o0, sem_o1,
                        sem_ob):
        nc = 2
        wid = lax.axis_index("s") * nc + lax.axis_index("c")
        pred_bufs = (pred_v0, pred_v1)
        out_bufs = (out_v0, out_v1)
        sem_in = (sem_i0, sem_i1)
        sem_out = (sem_o0, sem_o1)

        def in_cp(p, b0, buf):
            return pltpu.make_async_copy(
                pred_hbm.at[pl.ds(2 * p, 2), :, pl.ds(b0, _CHK)],
                pred_bufs[buf], sem_in[buf])

        def in_tail_cp(b0):
            return pltpu.make_async_copy(
                pred_hbm.at[pl.ds(_T - 1, 1), :, pl.ds(b0, _CHK)],
                pred_bufs[0].at[pl.ds(0, 1)], sem_in[0])

        def out_cp(t, b0, buf):
            return pltpu.make_async_copy(
                out_bufs[buf],
                out_hbm.at[pl.ds(t, 1), :, pl.ds(b0, _CHK)], sem_out[buf])

        def assemble(pred_buf, tsub, out_buf):
            @plsc.parallel_loop(0, _CHK, 16, unroll=3)
            def kbody(o):
                loaded = {}
                for c in range(_C):
                    kind, j = _SRC[c]
                    key = (kind, j)
                    if key not in loaded:
                        if kind == "pred":
                            loaded[key] = pred_buf[tsub, j, pl.ds(o, 16)]
                        else:
                            loaded[key] = obs_v[0, _OBS_ROWMAP[j],
                                                pl.ds(o, 16)]
                    out_buf[0, c, pl.ds(o, 16)] = loaded[key]

        def obs_cps(b0):
            return [
                pltpu.make_async_copy(
                    obs_hbm.at[pl.ds(_T * 2 - 1, 1), pl.ds(16 * g, 8),
                               pl.ds(b0, _CHK)],
                    obs_v.at[:, pl.ds(8 * g, 8)], sem_ob)
                for g in range(3)
            ]

        def chunk_base(ch):
            return wid * (_NCHUNK * _CHK) + ch * _CHK

        for cp in obs_cps(chunk_base(0)):
            cp.start()
        in_cp(0, chunk_base(0), 0).start()

        n_pairs = (_T - 1) // 2

        def chunk_body(ch, carry):
            b0 = chunk_base(ch)
            for cp in obs_cps(b0):
                cp.wait()

            def quad_body(pq, c2):
                p0 = 2 * pq
                in_cp(p0, b0, 0).wait()
                in_cp(p0 + 1, b0, 1).start()

                @pl.when(pq > 0)
                def _():
                    out_cp(2 * p0 - 2, b0, 0).wait()

                assemble(pred_bufs[0], 0, out_bufs[0])
                out_cp(2 * p0, b0, 0).start()

                @pl.when(pq > 0)
                def _():
                    out_cp(2 * p0 - 1, b0, 1).wait()

                assemble(pred_bufs[0], 1, out_bufs[1])
                out_cp(2 * p0 + 1, b0, 1).start()

                p1 = p0 + 1
                in_cp(p1, b0, 1).wait()

                @pl.when(pq < n_pairs // 2 - 1)
                def _():
                    in_cp(p1 + 1, b0, 0).start()

                @pl.when(pq == n_pairs // 2 - 1)
                def _():
                    in_tail_cp(b0).start()

                out_cp(2 * p1 - 2, b0, 0).wait()
                assemble(pred_bufs[1], 0, out_bufs[0])
                out_cp(2 * p1, b0, 0).start()
                out_cp(2 * p1 - 1, b0, 1).wait()
                assemble(pred_bufs[1], 1, out_bufs[1])
                out_cp(2 * p1 + 1, b0, 1).start()
                return c2

            lax.fori_loop(0, n_pairs // 2, quad_body, 0)
            t_last = _T - 1
            in_tail_cp(b0).wait()
            out_cp(t_last - 2, b0, 0).wait()
            assemble(pred_bufs[0], 0, out_bufs[0])
            out_cp(t_last, b0, 0).start()

            @pl.when(ch + 1 < _NCHUNK)
            def _():
                for cp in obs_cps(chunk_base(ch + 1)):
                    cp.start()
                in_cp(0, chunk_base(ch + 1), 0).start()

            out_cp(t_last - 1, b0, 1).wait()
            out_cp(t_last, b0, 0).wait()
            return carry

        lax.fori_loop(0, _NCHUNK, chunk_body, 0)

    return _sc_postprocess


def kernel(observed_pose, pred_pose):
    pred_r = jnp.transpose(pred_pose, (1, 2, 0))
    obs_r = jnp.transpose(observed_pose, (1, 2, 0))
    out_r = _make_sc_call()(obs_r, pred_r)
    return jnp.transpose(out_r, (2, 0, 1))
